# baseline (speedup 1.0000x reference)
import functools

import jax
import jax.numpy as jnp
from jax import lax
from jax.experimental import pallas as pl

D_MODEL = 1024
D_FF = 4096
H = 16
DH = 64
N_BUCKETS = 64
N_HASHES = 4
CHUNK = 64
GRP = 8
TOK_BLK = 512
TOK_BLK_C = 256


def _proj_body(x_ref, g_ref, b_ref, wqk_ref, wv_ref, rot_ref,
               qk_ref, v_ref, rt_ref):
    x = x_ref[...]
    m = jnp.mean(x, axis=-1, keepdims=True)
    xc = x - m
    var = jnp.mean(xc * xc, axis=-1, keepdims=True)
    xn = xc * lax.rsqrt(var + 1e-5) * g_ref[...] + b_ref[...]
    qk = jnp.dot(xn, wqk_ref[...], preferred_element_type=jnp.float32)
    qk_ref[...] = qk
    v_ref[...] = jnp.dot(xn, wv_ref[...], preferred_element_type=jnp.float32)
    rt_ref[...] = jnp.dot(qk, rot_ref[...], preferred_element_type=jnp.float32)


def _attn_body(q_ref, qp_ref, v_ref, vp_ref, br_ref, bp_ref, pr_ref, pp_ref,
               o_ref, lse_ref):
    qfull = q_ref[0]
    qprev = qp_ref[0]
    vfull = v_ref[0]
    vprev = vp_ref[0]
    brow = br_ref[0, 0]
    brow_p = bp_ref[0, 0]
    prow = pr_ref[0, 0]
    prow_p = pp_ref[0, 0]
    for j in range(GRP):
        lo = j * CHUNK
        qj = qfull[lo:lo + CHUNK, :DH]
        bq = qfull[lo:lo + CHUNK, DH:DH + 1]
        pq = qfull[lo:lo + CHUNK, DH + 1:DH + 2]
        if j == 0:
            kprev = qprev[(GRP - 1) * CHUNK:, :DH]
            vprevj = vprev[(GRP - 1) * CHUNK:]
            b_prev = brow_p[GRP - 1:GRP]
            p_prev = prow_p[GRP - 1:GRP]
        else:
            kprev = qfull[lo - CHUNK:lo, :DH]
            vprevj = vfull[lo - CHUNK:lo]
            b_prev = brow[j - 1:j]
            p_prev = prow[j - 1:j]
        kcat = jnp.concatenate([kprev, qfull[lo:lo + CHUNK, :DH]], axis=0)
        vcat = jnp.concatenate([vprevj, vfull[lo:lo + CHUNK]], axis=0)
        knorm = kcat * (1.0 / (jnp.sqrt(
            jnp.sum(kcat * kcat, axis=-1, keepdims=True)) + 1e-6))
        dots = lax.dot_general(qj, knorm, (((1,), (1,)), ((), ())),
                               preferred_element_type=jnp.float32)
        dots = dots * (1.0 / (float(DH) ** 0.5))
        b_e = jnp.concatenate([b_prev, brow[j:j + 1]], axis=1)
        p_e = jnp.concatenate([p_prev, prow[j:j + 1]], axis=1)
        dots = jnp.where(bq == b_e, dots, -1e9)
        dots = jnp.where(pq == p_e, -1e5, dots)
        m = jnp.max(dots, axis=-1, keepdims=True)
        e = jnp.exp(dots - m)
        s = jnp.sum(e, axis=-1, keepdims=True)
        o = jnp.dot(e, vcat, preferred_element_type=jnp.float32) / s
        o_ref[0, lo:lo + CHUNK, :] = o
        lse_ref[0, lo:lo + CHUNK, :] = m + jnp.log(s)


def _out_ff_body(o_ref, x1_ref, x2_ref, wo_ref, g_ref, b_ref,
                 w1_ref, b1_ref, w2_ref, b2_ref, y1_ref, y2_ref):
    y1 = x1_ref[...] + jnp.dot(o_ref[...], wo_ref[...],
                               preferred_element_type=jnp.float32)
    y1_ref[...] = y1
    m = jnp.mean(y1, axis=-1, keepdims=True)
    xc = y1 - m
    var = jnp.mean(xc * xc, axis=-1, keepdims=True)
    t = xc * lax.rsqrt(var + 1e-5) * g_ref[...] + b_ref[...]
    h = jax.nn.gelu(jnp.dot(t, w1_ref[...],
                            preferred_element_type=jnp.float32) + b1_ref[...])
    y2_ref[...] = x2_ref[...] + jnp.dot(h, w2_ref[...],
                                        preferred_element_type=jnp.float32) + b2_ref[...]


def _build_rotmat():
    rot = jax.random.normal(jax.random.key(42),
                            (N_HASHES, DH, N_BUCKETS // 2), dtype=jnp.float32)
    eye = jnp.eye(H, dtype=jnp.float32)
    blk = jnp.einsum('gh,rdn->rgdhn', eye, rot)
    return blk.transpose(1, 2, 0, 3, 4).reshape(D_MODEL, N_HASHES * H * 32)


def kernel(x1, x2, Wqk, Wv, Wo, W1, b1, W2, b2, ln1_g, ln1_b, ln2_g, ln2_b):
    B, S, _ = x1.shape
    nc = S // CHUNK
    ng = nc // GRP
    T = B * S
    nblk = T // TOK_BLK
    inst = N_HASHES * B * H

    rotmat = _build_rotmat()
    x2f = x2.reshape(T, D_MODEL)

    row = lambda a: a.reshape(1, -1)
    full = lambda r, c: pl.BlockSpec((r, c), lambda i: (0, 0))
    qk, v, rt = pl.pallas_call(
        _proj_body,
        grid=(nblk,),
        in_specs=[
            pl.BlockSpec((TOK_BLK, D_MODEL), lambda i: (i, 0)),
            full(1, D_MODEL), full(1, D_MODEL),
            full(D_MODEL, D_MODEL), full(D_MODEL, D_MODEL),
            full(D_MODEL, N_HASHES * H * 32),
        ],
        out_specs=[
            pl.BlockSpec((TOK_BLK, D_MODEL), lambda i: (i, 0)),
            pl.BlockSpec((TOK_BLK, D_MODEL), lambda i: (i, 0)),
            pl.BlockSpec((TOK_BLK, N_HASHES * H * 32), lambda i: (i, 0)),
        ],
        out_shape=[
            jax.ShapeDtypeStruct((T, D_MODEL), jnp.float32),
            jax.ShapeDtypeStruct((T, D_MODEL), jnp.float32),
            jax.ShapeDtypeStruct((T, N_HASHES * H * 32), jnp.float32),
        ],
    )(x2f, row(ln1_g), row(ln1_b), Wqk, Wv, rotmat)

    rt = rt.reshape(B, S, N_HASHES, H, 32)
    rt = jnp.concatenate([rt, -rt], axis=-1)
    buckets = jnp.argmax(rt, axis=-1).astype(jnp.int32)
    buckets = buckets.transpose(2, 0, 3, 1)
    pos = jnp.arange(S, dtype=jnp.int32)
    skey = buckets * S + pos[None, None, None, :]
    pos_b = jnp.broadcast_to(pos, skey.shape)
    skey_s, perm = lax.sort((skey, pos_b), dimension=3, num_keys=1)
    sb = skey_s // S
    inv = jnp.put_along_axis(jnp.zeros_like(perm), perm, pos_b, axis=-1,
                             inplace=False)

    bh_base = (jnp.arange(B, dtype=jnp.int32)[:, None] * S) * H \
        + jnp.arange(H, dtype=jnp.int32)[None, :]
    g_idx = bh_base[None, :, :, None] + perm * H
    sqk = jnp.take(qk.reshape(T * H, DH), g_idx.reshape(-1), axis=0)
    sv = jnp.take(v.reshape(T * H, DH), g_idx.reshape(-1), axis=0)
    sqk = sqk.reshape(N_HASHES, B, H, S, DH)
    sv = sv.reshape(N_HASHES, B, H, S, DH)

    sbf = sb.astype(jnp.float32)
    spf = perm.astype(jnp.float32)
    a_q = jnp.concatenate([sqk, sbf[..., None], spf[..., None]], axis=-1)
    a_q = a_q.reshape(inst, S, DH + 2)
    a_v = sv.reshape(inst, S, DH)
    b_row = sbf.reshape(inst, ng, GRP, CHUNK)
    p_row = spf.reshape(inst, ng, GRP, CHUNK)

    o_s, lse_s = pl.pallas_call(
        _attn_body,
        grid=(inst, ng),
        in_specs=[
            pl.BlockSpec((1, GRP * CHUNK, DH + 2), lambda i, g: (i, g, 0)),
            pl.BlockSpec((1, GRP * CHUNK, DH + 2),
                         lambda i, g: (i, (g + ng - 1) % ng, 0)),
            pl.BlockSpec((1, GRP * CHUNK, DH), lambda i, g: (i, g, 0)),
            pl.BlockSpec((1, GRP * CHUNK, DH),
                         lambda i, g: (i, (g + ng - 1) % ng, 0)),
            pl.BlockSpec((1, 1, GRP, CHUNK), lambda i, g: (i, g, 0, 0)),
            pl.BlockSpec((1, 1, GRP, CHUNK),
                         lambda i, g: (i, (g + ng - 1) % ng, 0, 0)),
            pl.BlockSpec((1, 1, GRP, CHUNK), lambda i, g: (i, g, 0, 0)),
            pl.BlockSpec((1, 1, GRP, CHUNK),
                         lambda i, g: (i, (g + ng - 1) % ng, 0, 0)),
        ],
        out_specs=[
            pl.BlockSpec((1, GRP * CHUNK, DH), lambda i, g: (i, g, 0)),
            pl.BlockSpec((1, GRP * CHUNK, 1), lambda i, g: (i, g, 0)),
        ],
        out_shape=[
            jax.ShapeDtypeStruct((inst, S, DH), jnp.float32),
            jax.ShapeDtypeStruct((inst, S, 1), jnp.float32),
        ],
    )(a_q, a_q, a_v, a_v, b_row, b_row, p_row, p_row)

    rbh_base = (jnp.arange(N_HASHES * B * H, dtype=jnp.int32)
                .reshape(N_HASHES, B, H) * S)
    u_idx = rbh_base[..., None] + inv
    u_idx = u_idx.transpose(0, 1, 3, 2)
    o_all = jnp.take(o_s.reshape(inst * S, DH),
                     u_idx.reshape(-1), axis=0).reshape(N_HASHES, T, H, DH)
    lse_all = jnp.take(lse_s.reshape(inst * S), u_idx.reshape(-1),
                       axis=0).reshape(N_HASHES, T, H)
    w = jax.nn.softmax(lse_all, axis=0)[..., None]
    o_comb = jnp.sum(o_all * w, axis=0).reshape(T, D_MODEL)

    y1, y2 = pl.pallas_call(
        _out_ff_body,
        grid=(T // TOK_BLK_C,),
        in_specs=[
            pl.BlockSpec((TOK_BLK_C, D_MODEL), lambda i: (i, 0)),
            pl.BlockSpec((TOK_BLK_C, D_MODEL), lambda i: (i, 0)),
            pl.BlockSpec((TOK_BLK_C, D_MODEL), lambda i: (i, 0)),
            full(D_MODEL, D_MODEL),
            full(1, D_MODEL), full(1, D_MODEL),
            full(D_MODEL, D_FF), full(1, D_FF),
            full(D_FF, D_MODEL), full(1, D_MODEL),
        ],
        out_specs=[
            pl.BlockSpec((TOK_BLK_C, D_MODEL), lambda i: (i, 0)),
            pl.BlockSpec((TOK_BLK_C, D_MODEL), lambda i: (i, 0)),
        ],
        out_shape=[
            jax.ShapeDtypeStruct((T, D_MODEL), jnp.float32),
            jax.ShapeDtypeStruct((T, D_MODEL), jnp.float32),
        ],
    )(o_comb, x1.reshape(T, D_MODEL), x2f, Wo, row(ln2_g), row(ln2_b),
      W1, row(b1), W2, row(b2))

    return (y1.reshape(B, S, D_MODEL), y2.reshape(B, S, D_MODEL))


# speedup vs baseline: 4.9351x; 4.9351x over previous
import functools

import jax
import jax.numpy as jnp
from jax import lax
from jax.experimental import pallas as pl
from jax.experimental.pallas import tpu as pltpu
from jax.experimental.pallas import tpu_sc as plsc

D_MODEL = 1024
D_FF = 4096
H = 16
DH = 64
N_BUCKETS = 64
N_HASHES = 4
CHUNK = 64
GRP = 8
TOK_BLK = 512
TOK_BLK_C = 256


def _proj_body(x_ref, g_ref, b_ref, wqk_ref, wv_ref, rot_ref,
               qkv_ref, rt_ref):
    x = x_ref[...]
    m = jnp.mean(x, axis=-1, keepdims=True)
    xc = x - m
    var = jnp.mean(xc * xc, axis=-1, keepdims=True)
    xn = xc * lax.rsqrt(var + 1e-5) * g_ref[...] + b_ref[...]
    qk = jnp.dot(xn, wqk_ref[...], preferred_element_type=jnp.float32)
    v = jnp.dot(xn, wv_ref[...], preferred_element_type=jnp.float32)
    rt_ref[...] = jnp.dot(qk, rot_ref[...], preferred_element_type=jnp.float32)
    parts = []
    for h in range(H):
        parts.append(qk[:, h * DH:(h + 1) * DH])
        parts.append(v[:, h * DH:(h + 1) * DH])
    qkv_ref[...] = jnp.concatenate(parts, axis=1)


def _csort_body(b_ref, inv_ref, offr_ref, offc_ref):
    bcol = b_ref[0]
    s_len = bcol.shape[0]
    iota_row = lax.broadcasted_iota(jnp.int32, (1, N_BUCKETS), 1).astype(jnp.float32)
    onehot = (bcol == iota_row).astype(jnp.float32)
    pre_incl = onehot
    k = 1
    while k < s_len:
        shifted = jnp.concatenate(
            [jnp.zeros((k, N_BUCKETS), jnp.float32), pre_incl[:s_len - k]],
            axis=0)
        pre_incl = pre_incl + shifted
        k *= 2
    pre = pre_incl - onehot
    tot_row = pre_incl[s_len - 1:s_len, :]
    bi0 = lax.broadcasted_iota(jnp.int32, (N_BUCKETS, N_BUCKETS), 0)
    bi1 = lax.broadcasted_iota(jnp.int32, (N_BUCKETS, N_BUCKETS), 1)
    upper = (bi0 < bi1).astype(jnp.float32)
    off_row = jnp.dot(tot_row, upper, preferred_element_type=jnp.float32)
    tot_col = lax.dot_general(onehot, jnp.ones((s_len, 1), jnp.float32),
                              (((0,), (0,)), ((), ())),
                              preferred_element_type=jnp.float32)
    off_col = jnp.dot((bi0 > bi1).astype(jnp.float32), tot_col,
                      preferred_element_type=jnp.float32)
    inv_ref[0] = jnp.sum(onehot * (pre + off_row), axis=-1, keepdims=True)
    offr_ref[0] = off_row
    offc_ref[0] = off_col


def _attn_body(qv_ref, qvp_ref, offr_ref, offc_ref, o_ref):
    qvfull = qv_ref[0]
    qvprev = qvp_ref[0]
    off_row = offr_ref[0]
    off_col = offc_ref[0]
    g = pl.program_id(1)
    nc_total = pl.num_programs(1) * GRP
    col_i = lax.broadcasted_iota(jnp.int32, (1, CHUNK), 1).astype(jnp.float32)
    row_i = lax.broadcasted_iota(jnp.int32, (CHUNK, 1), 0).astype(jnp.float32)
    ri2 = lax.broadcasted_iota(jnp.int32, (CHUNK, 2 * CHUNK), 0)
    ci2 = lax.broadcasted_iota(jnp.int32, (CHUNK, 2 * CHUNK), 1)
    self_mask = ci2 == ri2 + CHUNK
    for j in range(GRP):
        lo = j * CHUNK
        cidx = g * GRP + j
        pidx = lax.rem(cidx + nc_total - 1, nc_total)
        base = (cidx * CHUNK).astype(jnp.float32)
        pbase = (pidx * CHUNK).astype(jnp.float32)
        bq = jnp.sum((base + row_i >= off_row).astype(jnp.float32),
                     axis=-1, keepdims=True) - 1.0
        b_cur = jnp.sum((base + col_i >= off_col).astype(jnp.float32),
                        axis=0, keepdims=True) - 1.0
        b_prev = jnp.sum((pbase + col_i >= off_col).astype(jnp.float32),
                         axis=0, keepdims=True) - 1.0
        qj = qvfull[lo:lo + CHUNK, :DH]
        if j == 0:
            prev_qv = qvprev[(GRP - 1) * CHUNK:]
        else:
            prev_qv = qvfull[lo - CHUNK:lo]
        kcat = jnp.concatenate([prev_qv[:, :DH],
                                qvfull[lo:lo + CHUNK, :DH]], axis=0)
        vcat = jnp.concatenate([prev_qv[:, DH:],
                                qvfull[lo:lo + CHUNK, DH:]], axis=0)
        knorm = kcat * (1.0 / (jnp.sqrt(
            jnp.sum(kcat * kcat, axis=-1, keepdims=True)) + 1e-6))
        dots = lax.dot_general(qj, knorm, (((1,), (1,)), ((), ())),
                               preferred_element_type=jnp.float32)
        dots = dots * (1.0 / (float(DH) ** 0.5))
        b_e = jnp.concatenate([b_prev, b_cur], axis=1)
        dots = jnp.where(bq == b_e, dots, -1e9)
        dots = jnp.where(self_mask, -1e5, dots)
        m = jnp.max(dots, axis=-1, keepdims=True)
        e = jnp.exp(dots - m)
        s = jnp.sum(e, axis=-1, keepdims=True)
        o = jnp.dot(e, vcat, preferred_element_type=jnp.float32) / s
        lse = m + jnp.log(s)
        o_ref[0, lo:lo + CHUNK, :] = jnp.concatenate(
            [o, lse, jnp.zeros((CHUNK, DH - 1), jnp.float32)], axis=1)


def _out_ff_body(o_ref, x1_ref, x2_ref, wo_ref, g_ref, b_ref,
                 w1_ref, b1_ref, w2_ref, b2_ref, y1_ref, y2_ref):
    y1 = x1_ref[...] + jnp.dot(o_ref[...], wo_ref[...],
                               preferred_element_type=jnp.float32)
    y1_ref[...] = y1
    m = jnp.mean(y1, axis=-1, keepdims=True)
    xc = y1 - m
    var = jnp.mean(xc * xc, axis=-1, keepdims=True)
    t = xc * lax.rsqrt(var + 1e-5) * g_ref[...] + b_ref[...]
    h = jax.nn.gelu(jnp.dot(t, w1_ref[...],
                            preferred_element_type=jnp.float32) + b1_ref[...])
    y2_ref[...] = x2_ref[...] + jnp.dot(h, w2_ref[...],
                                        preferred_element_type=jnp.float32) + b2_ref[...]


_SC_NC, _SC_NS = 2, 16
_SC_NW = _SC_NC * _SC_NS
_GCHK = 128


def _sc_gather(tables, idx):
    M = idx.shape[0]
    W = tables[0].shape[1]
    per_w = M // _SC_NW
    n_it = per_w // _GCHK
    nt = len(tables)
    mesh = plsc.VectorSubcoreMesh(core_axis_name="c", subcore_axis_name="s",
                                  num_cores=_SC_NC, num_subcores=_SC_NS)

    @functools.partial(
        pl.kernel, mesh=mesh,
        out_type=[jax.ShapeDtypeStruct((M, W), jnp.float32)
                  for _ in range(nt)],
        scratch_types=(
            [pltpu.VMEM((_GCHK,), jnp.int32)]
            + [pltpu.VMEM((_GCHK, W), jnp.float32) for _ in range(nt)]
            + [pltpu.SemaphoreType.DMA for _ in range(nt)]
        ),
    )
    def gk(*refs):
        t_hbm = refs[:nt]
        idx_hbm = refs[nt]
        o_hbm = refs[nt + 1:nt + 1 + nt]
        idx_v = refs[nt + 1 + nt]
        rows = refs[nt + 2 + nt:nt + 2 + 2 * nt]
        sems = refs[nt + 2 + 2 * nt:]
        wid = lax.axis_index("s") * _SC_NC + lax.axis_index("c")
        base = wid * per_w

        def body(j, carry):
            off = base + j * _GCHK
            pltpu.sync_copy(idx_hbm.at[pl.ds(off, _GCHK)], idx_v)
            cps = [pltpu.async_copy(t_hbm[i].at[idx_v], rows[i], sems[i])
                   for i in range(nt)]
            for c in cps:
                c.wait()
            for i in range(nt):
                pltpu.sync_copy(rows[i], o_hbm[i].at[pl.ds(off, _GCHK)])
            return carry

        lax.fori_loop(0, n_it, body, 0)

    out = gk(*tables, idx)
    return list(out) if isinstance(out, (list, tuple)) else [out]


def _sc_scatter(src, idx3):
    W = src.shape[1]
    n_src = src.shape[0]
    n_it = idx3.shape[1]
    M = _SC_NW * n_it * _GCHK
    per_w = n_it * _GCHK
    mesh = plsc.VectorSubcoreMesh(core_axis_name="c", subcore_axis_name="s",
                                  num_cores=_SC_NC, num_subcores=_SC_NS)

    @functools.partial(
        pl.kernel, mesh=mesh,
        out_type=jax.ShapeDtypeStruct((M, W), jnp.float32),
        scratch_types=[
            pltpu.VMEM((_GCHK,), jnp.int32),
            pltpu.VMEM((_GCHK, W), jnp.float32),
            pltpu.SemaphoreType.DMA,
        ],
    )
    def sk(src_hbm, idx_hbm, out_hbm, idx_v, rows_v, sem):
        wid = lax.axis_index("s") * _SC_NC + lax.axis_index("c")
        base = wid * per_w

        def body(j, carry):
            src_off = lax.rem(base + j * _GCHK, n_src)
            pltpu.sync_copy(src_hbm.at[pl.ds(src_off, _GCHK)], rows_v)
            pltpu.sync_copy(idx_hbm.at[wid, j], idx_v)
            pltpu.async_copy(rows_v, out_hbm.at[idx_v], sem).wait()
            return carry

        lax.fori_loop(0, n_it, body, 0)

    return sk(src, idx3)


def _build_rotmat():
    rot = jax.random.normal(jax.random.key(42),
                            (N_HASHES, DH, N_BUCKETS // 2), dtype=jnp.float32)
    eye = jnp.eye(H, dtype=jnp.float32)
    blk = jnp.einsum('gh,rdn->rgdhn', eye, rot)
    return blk.transpose(1, 2, 0, 3, 4).reshape(D_MODEL, N_HASHES * H * 32)


def kernel(x1, x2, Wqk, Wv, Wo, W1, b1, W2, b2, ln1_g, ln1_b, ln2_g, ln2_b):
    B, S, _ = x1.shape
    nc = S // CHUNK
    ng = nc // GRP
    T = B * S
    nblk = T // TOK_BLK
    inst = N_HASHES * B * H

    rotmat = _build_rotmat()
    x2f = x2.reshape(T, D_MODEL)

    row = lambda a: a.reshape(1, -1)
    full = lambda r, c: pl.BlockSpec((r, c), lambda i: (0, 0))
    qkv, rt = pl.pallas_call(
        _proj_body,
        grid=(nblk,),
        in_specs=[
            pl.BlockSpec((TOK_BLK, D_MODEL), lambda i: (i, 0)),
            full(1, D_MODEL), full(1, D_MODEL),
            full(D_MODEL, D_MODEL), full(D_MODEL, D_MODEL),
            full(D_MODEL, N_HASHES * H * 32),
        ],
        out_specs=[
            pl.BlockSpec((TOK_BLK, 2 * D_MODEL), lambda i: (i, 0)),
            pl.BlockSpec((TOK_BLK, N_HASHES * H * 32), lambda i: (i, 0)),
        ],
        out_shape=[
            jax.ShapeDtypeStruct((T, 2 * D_MODEL), jnp.float32),
            jax.ShapeDtypeStruct((T, N_HASHES * H * 32), jnp.float32),
        ],
    )(x2f, row(ln1_g), row(ln1_b), Wqk, Wv, rotmat)

    rt = rt.reshape(B, S, N_HASHES, H, 32)
    rt = jnp.concatenate([rt, -rt], axis=-1)
    buckets = jnp.argmax(rt, axis=-1).astype(jnp.int32)
    bcol = (buckets.transpose(2, 0, 3, 1)
            .reshape(inst, S, 1).astype(jnp.float32))

    inv_col, off_row, off_col = pl.pallas_call(
        _csort_body,
        grid=(inst,),
        in_specs=[pl.BlockSpec((1, S, 1), lambda i: (i, 0, 0))],
        out_specs=[
            pl.BlockSpec((1, S, 1), lambda i: (i, 0, 0)),
            pl.BlockSpec((1, 1, N_BUCKETS), lambda i: (i, 0, 0)),
            pl.BlockSpec((1, N_BUCKETS, 1), lambda i: (i, 0, 0)),
        ],
        out_shape=[
            jax.ShapeDtypeStruct((inst, S, 1), jnp.float32),
            jax.ShapeDtypeStruct((inst, 1, N_BUCKETS), jnp.float32),
            jax.ShapeDtypeStruct((inst, N_BUCKETS, 1), jnp.float32),
        ],
    )(bcol)
    inv = inv_col.reshape(N_HASHES, B, H, S).astype(jnp.int32)

    rbh_base = (jnp.arange(N_HASHES * B * H, dtype=jnp.int32)
                .reshape(N_HASHES, B, H) * S)
    u_idx = (rbh_base[..., None] + inv).transpose(0, 1, 3, 2)
    a_qv = _sc_scatter(qkv.reshape(T * H, 2 * DH),
                       u_idx.reshape(_SC_NW, -1, _GCHK))
    a_qv = a_qv.reshape(inst, S, 2 * DH)

    (ol_s,) = pl.pallas_call(
        _attn_body,
        grid=(inst, ng),
        in_specs=[
            pl.BlockSpec((1, GRP * CHUNK, 2 * DH), lambda i, g: (i, g, 0)),
            pl.BlockSpec((1, GRP * CHUNK, 2 * DH),
                         lambda i, g: (i, (g + ng - 1) % ng, 0)),
            pl.BlockSpec((1, 1, N_BUCKETS), lambda i, g: (i, 0, 0)),
            pl.BlockSpec((1, N_BUCKETS, 1), lambda i, g: (i, 0, 0)),
        ],
        out_specs=[
            pl.BlockSpec((1, GRP * CHUNK, 2 * DH), lambda i, g: (i, g, 0)),
        ],
        out_shape=[
            jax.ShapeDtypeStruct((inst, S, 2 * DH), jnp.float32),
        ],
    )(a_qv, a_qv, off_row, off_col)

    (olr,) = _sc_gather([ol_s.reshape(inst * S, 2 * DH)], u_idx.reshape(-1))
    olr = olr.reshape(N_HASHES, T, H, 2 * DH)
    o_all = olr[..., :DH]
    lse_all = olr[..., DH]
    w = jax.nn.softmax(lse_all, axis=0)[..., None]
    o_comb = jnp.sum(o_all * w, axis=0).reshape(T, D_MODEL)

    y1, y2 = pl.pallas_call(
        _out_ff_body,
        grid=(T // TOK_BLK_C,),
        in_specs=[
            pl.BlockSpec((TOK_BLK_C, D_MODEL), lambda i: (i, 0)),
            pl.BlockSpec((TOK_BLK_C, D_MODEL), lambda i: (i, 0)),
            pl.BlockSpec((TOK_BLK_C, D_MODEL), lambda i: (i, 0)),
            full(D_MODEL, D_MODEL),
            full(1, D_MODEL), full(1, D_MODEL),
            full(D_MODEL, D_FF), full(1, D_FF),
            full(D_FF, D_MODEL), full(1, D_MODEL),
        ],
        out_specs=[
            pl.BlockSpec((TOK_BLK_C, D_MODEL), lambda i: (i, 0)),
            pl.BlockSpec((TOK_BLK_C, D_MODEL), lambda i: (i, 0)),
        ],
        out_shape=[
            jax.ShapeDtypeStruct((T, D_MODEL), jnp.float32),
            jax.ShapeDtypeStruct((T, D_MODEL), jnp.float32),
        ],
    )(o_comb, x1.reshape(T, D_MODEL), x2f, Wo, row(ln2_g), row(ln2_b),
      W1, row(b1), W2, row(b2))

    return (y1.reshape(B, S, D_MODEL), y2.reshape(B, S, D_MODEL))


# speedup vs baseline: 6.4338x; 1.3037x over previous
import functools

import jax
import jax.numpy as jnp
from jax import lax
from jax.experimental import pallas as pl
from jax.experimental.pallas import tpu as pltpu
from jax.experimental.pallas import tpu_sc as plsc

D_MODEL = 1024
D_FF = 4096
H = 16
DH = 64
N_BUCKETS = 64
N_HASHES = 4
CHUNK = 64
GRP = 8
TOK_BLK = 512
TOK_BLK_C = 256


def _proj_body(x_ref, g_ref, b_ref, wqk_ref, wv_ref, rot_ref,
               qkv_ref, rt_ref):
    x = x_ref[...]
    m = jnp.mean(x, axis=-1, keepdims=True)
    xc = x - m
    var = jnp.mean(xc * xc, axis=-1, keepdims=True)
    xn = xc * lax.rsqrt(var + 1e-5) * g_ref[...] + b_ref[...]
    qk = jnp.dot(xn, wqk_ref[...], preferred_element_type=jnp.float32)
    v = jnp.dot(xn, wv_ref[...], preferred_element_type=jnp.float32)
    rt_ref[...] = jnp.dot(qk, rot_ref[...], preferred_element_type=jnp.float32)
    parts = []
    for h in range(H):
        parts.append(qk[:, h * DH:(h + 1) * DH])
        parts.append(v[:, h * DH:(h + 1) * DH])
    qkv_ref[...] = jnp.concatenate(parts, axis=1)


def _csort_body(b_ref, inv_ref, offr_ref, offc_ref):
    bcol = b_ref[0]
    s_len = bcol.shape[0]
    iota_row = lax.broadcasted_iota(jnp.int32, (1, N_BUCKETS), 1).astype(jnp.float32)
    onehot = (bcol == iota_row).astype(jnp.float32)
    pre_incl = onehot
    k = 1
    while k < s_len:
        shifted = jnp.concatenate(
            [jnp.zeros((k, N_BUCKETS), jnp.float32), pre_incl[:s_len - k]],
            axis=0)
        pre_incl = pre_incl + shifted
        k *= 2
    pre = pre_incl - onehot
    tot_row = pre_incl[s_len - 1:s_len, :]
    bi0 = lax.broadcasted_iota(jnp.int32, (N_BUCKETS, N_BUCKETS), 0)
    bi1 = lax.broadcasted_iota(jnp.int32, (N_BUCKETS, N_BUCKETS), 1)
    upper = (bi0 < bi1).astype(jnp.float32)
    off_row = jnp.dot(tot_row, upper, preferred_element_type=jnp.float32)
    tot_col = lax.dot_general(onehot, jnp.ones((s_len, 1), jnp.float32),
                              (((0,), (0,)), ((), ())),
                              preferred_element_type=jnp.float32)
    off_col = jnp.dot((bi0 > bi1).astype(jnp.float32), tot_col,
                      preferred_element_type=jnp.float32)
    inv_ref[0] = jnp.sum(onehot * (pre + off_row), axis=-1, keepdims=True)
    offr_ref[0] = off_row
    offc_ref[0] = off_col


def _attn_body(qv_ref, qvp_ref, offr_ref, offc_ref, o_ref):
    qvfull = qv_ref[0]
    qvprev = qvp_ref[0]
    off_row = offr_ref[0]
    off_col = offc_ref[0]
    g = pl.program_id(1)
    nc_total = pl.num_programs(1) * GRP
    W = GRP * CHUNK
    KW = W + CHUNK
    base0 = g * W
    pidx = lax.rem(g * GRP + nc_total - 1, nc_total)
    qv_ext = jnp.concatenate([qvprev[(GRP - 1) * CHUNK:], qvfull], axis=0)
    k = qv_ext[:, :DH]
    knorm = k * (1.0 / (jnp.sqrt(
        jnp.sum(k * k, axis=-1, keepdims=True)) + 1e-6))
    q = qvfull[:, :DH]
    dots = lax.dot_general(q, knorm, (((1,), (1,)), ((), ())),
                           preferred_element_type=jnp.float32)
    dots = dots * (1.0 / (float(DH) ** 0.5))
    ri = lax.broadcasted_iota(jnp.int32, (W, KW), 0)
    ci = lax.broadcasted_iota(jnp.int32, (W, KW), 1)
    kc = lax.div(ci, CHUNK)
    qc = lax.div(ri, CHUNK)
    band = (kc >= qc) & (kc <= qc + 1)
    self_m = ci == ri + CHUNK
    qslot = (base0 + lax.broadcasted_iota(jnp.int32, (W, 1), 0)
             ).astype(jnp.float32)
    bq = jnp.sum((qslot >= off_row).astype(jnp.float32),
                 axis=-1, keepdims=True)
    ci_row = lax.broadcasted_iota(jnp.int32, (1, KW), 1)
    kslot = jnp.where(ci_row < CHUNK, pidx * CHUNK + ci_row,
                      base0 + ci_row - CHUNK).astype(jnp.float32)
    b_e = jnp.sum((kslot >= off_col).astype(jnp.float32),
                  axis=0, keepdims=True)
    dots = jnp.where(band & (bq == b_e), dots, -1e9)
    dots = jnp.where(self_m, -1e5, dots)
    m = jnp.max(dots, axis=-1, keepdims=True)
    e = jnp.exp(dots - m)
    s = jnp.sum(e, axis=-1, keepdims=True)
    o = jnp.dot(e, qv_ext[:, DH:],
                preferred_element_type=jnp.float32) / s
    lse = m + jnp.log(s)
    o_ref[0] = jnp.concatenate(
        [o, lse, jnp.zeros((W, DH - 1), jnp.float32)], axis=1)


def _out_ff_body(o_ref, x1_ref, x2_ref, wo_ref, g_ref, b_ref,
                 w1_ref, b1_ref, w2_ref, b2_ref, y1_ref, y2_ref):
    y1 = x1_ref[...] + jnp.dot(o_ref[...], wo_ref[...],
                               preferred_element_type=jnp.float32)
    y1_ref[...] = y1
    m = jnp.mean(y1, axis=-1, keepdims=True)
    xc = y1 - m
    var = jnp.mean(xc * xc, axis=-1, keepdims=True)
    t = xc * lax.rsqrt(var + 1e-5) * g_ref[...] + b_ref[...]
    h = jax.nn.gelu(jnp.dot(t, w1_ref[...],
                            preferred_element_type=jnp.float32) + b1_ref[...])
    y2_ref[...] = x2_ref[...] + jnp.dot(h, w2_ref[...],
                                        preferred_element_type=jnp.float32) + b2_ref[...]


_SC_NC, _SC_NS = 2, 16
_SC_NW = _SC_NC * _SC_NS
_GCHK = 128


def _sc_gather(tables, idx):
    M = idx.shape[0]
    W = tables[0].shape[1]
    per_w = M // _SC_NW
    n_it = per_w // _GCHK
    nt = len(tables)
    mesh = plsc.VectorSubcoreMesh(core_axis_name="c", subcore_axis_name="s",
                                  num_cores=_SC_NC, num_subcores=_SC_NS)

    @functools.partial(
        pl.kernel, mesh=mesh,
        out_type=[jax.ShapeDtypeStruct((M, W), jnp.float32)
                  for _ in range(nt)],
        scratch_types=(
            [pltpu.VMEM((_GCHK,), jnp.int32)]
            + [pltpu.VMEM((_GCHK, W), jnp.float32) for _ in range(nt)]
            + [pltpu.SemaphoreType.DMA for _ in range(nt)]
        ),
    )
    def gk(*refs):
        t_hbm = refs[:nt]
        idx_hbm = refs[nt]
        o_hbm = refs[nt + 1:nt + 1 + nt]
        idx_v = refs[nt + 1 + nt]
        rows = refs[nt + 2 + nt:nt + 2 + 2 * nt]
        sems = refs[nt + 2 + 2 * nt:]
        wid = lax.axis_index("s") * _SC_NC + lax.axis_index("c")
        base = wid * per_w

        def body(j, carry):
            off = base + j * _GCHK
            pltpu.sync_copy(idx_hbm.at[pl.ds(off, _GCHK)], idx_v)
            cps = [pltpu.async_copy(t_hbm[i].at[idx_v], rows[i], sems[i])
                   for i in range(nt)]
            for c in cps:
                c.wait()
            for i in range(nt):
                pltpu.sync_copy(rows[i], o_hbm[i].at[pl.ds(off, _GCHK)])
            return carry

        lax.fori_loop(0, n_it, body, 0)

    out = gk(*tables, idx)
    return list(out) if isinstance(out, (list, tuple)) else [out]


def _sc_scatter(src, idx3):
    W = src.shape[1]
    n_src = src.shape[0]
    n_it = idx3.shape[1]
    M = _SC_NW * n_it * _GCHK
    per_w = n_it * _GCHK
    mesh = plsc.VectorSubcoreMesh(core_axis_name="c", subcore_axis_name="s",
                                  num_cores=_SC_NC, num_subcores=_SC_NS)

    @functools.partial(
        pl.kernel, mesh=mesh,
        out_type=jax.ShapeDtypeStruct((M, W), jnp.float32),
        scratch_types=[
            pltpu.VMEM((_GCHK,), jnp.int32),
            pltpu.VMEM((_GCHK, W), jnp.float32),
            pltpu.SemaphoreType.DMA,
        ],
    )
    def sk(src_hbm, idx_hbm, out_hbm, idx_v, rows_v, sem):
        wid = lax.axis_index("s") * _SC_NC + lax.axis_index("c")
        base = wid * per_w

        def body(j, carry):
            src_off = lax.rem(base + j * _GCHK, n_src)
            pltpu.sync_copy(src_hbm.at[pl.ds(src_off, _GCHK)], rows_v)
            pltpu.sync_copy(idx_hbm.at[wid, j], idx_v)
            pltpu.async_copy(rows_v, out_hbm.at[idx_v], sem).wait()
            return carry

        lax.fori_loop(0, n_it, body, 0)

    return sk(src, idx3)


def _build_rotmat():
    rot = jax.random.normal(jax.random.key(42),
                            (N_HASHES, DH, N_BUCKETS // 2), dtype=jnp.float32)
    eye = jnp.eye(H, dtype=jnp.float32)
    blk = jnp.einsum('gh,rdn->rgdhn', eye, rot)
    return blk.transpose(1, 2, 0, 3, 4).reshape(D_MODEL, N_HASHES * H * 32)


def kernel(x1, x2, Wqk, Wv, Wo, W1, b1, W2, b2, ln1_g, ln1_b, ln2_g, ln2_b):
    B, S, _ = x1.shape
    nc = S // CHUNK
    ng = nc // GRP
    T = B * S
    nblk = T // TOK_BLK
    inst = N_HASHES * B * H

    rotmat = _build_rotmat()
    x2f = x2.reshape(T, D_MODEL)

    row = lambda a: a.reshape(1, -1)
    full = lambda r, c: pl.BlockSpec((r, c), lambda i: (0, 0))
    qkv, rt = pl.pallas_call(
        _proj_body,
        grid=(nblk,),
        in_specs=[
            pl.BlockSpec((TOK_BLK, D_MODEL), lambda i: (i, 0)),
            full(1, D_MODEL), full(1, D_MODEL),
            full(D_MODEL, D_MODEL), full(D_MODEL, D_MODEL),
            full(D_MODEL, N_HASHES * H * 32),
        ],
        out_specs=[
            pl.BlockSpec((TOK_BLK, 2 * D_MODEL), lambda i: (i, 0)),
            pl.BlockSpec((TOK_BLK, N_HASHES * H * 32), lambda i: (i, 0)),
        ],
        out_shape=[
            jax.ShapeDtypeStruct((T, 2 * D_MODEL), jnp.float32),
            jax.ShapeDtypeStruct((T, N_HASHES * H * 32), jnp.float32),
        ],
    )(x2f, row(ln1_g), row(ln1_b), Wqk, Wv, rotmat)

    rt = rt.reshape(B, S, N_HASHES, H, 32)
    rt = jnp.concatenate([rt, -rt], axis=-1)
    buckets = jnp.argmax(rt, axis=-1).astype(jnp.int32)
    bcol = (buckets.transpose(2, 0, 3, 1)
            .reshape(inst, S, 1).astype(jnp.float32))

    inv_col, off_row, off_col = pl.pallas_call(
        _csort_body,
        grid=(inst,),
        in_specs=[pl.BlockSpec((1, S, 1), lambda i: (i, 0, 0))],
        out_specs=[
            pl.BlockSpec((1, S, 1), lambda i: (i, 0, 0)),
            pl.BlockSpec((1, 1, N_BUCKETS), lambda i: (i, 0, 0)),
            pl.BlockSpec((1, N_BUCKETS, 1), lambda i: (i, 0, 0)),
        ],
        out_shape=[
            jax.ShapeDtypeStruct((inst, S, 1), jnp.float32),
            jax.ShapeDtypeStruct((inst, 1, N_BUCKETS), jnp.float32),
            jax.ShapeDtypeStruct((inst, N_BUCKETS, 1), jnp.float32),
        ],
    )(bcol)
    inv = inv_col.reshape(N_HASHES, B, H, S).astype(jnp.int32)

    rbh_base = (jnp.arange(N_HASHES * B * H, dtype=jnp.int32)
                .reshape(N_HASHES, B, H) * S)
    u_idx = (rbh_base[..., None] + inv).transpose(0, 1, 3, 2)
    a_qv = _sc_scatter(qkv.reshape(T * H, 2 * DH),
                       u_idx.reshape(_SC_NW, -1, _GCHK))
    a_qv = a_qv.reshape(inst, S, 2 * DH)

    (ol_s,) = pl.pallas_call(
        _attn_body,
        grid=(inst, ng),
        in_specs=[
            pl.BlockSpec((1, GRP * CHUNK, 2 * DH), lambda i, g: (i, g, 0)),
            pl.BlockSpec((1, GRP * CHUNK, 2 * DH),
                         lambda i, g: (i, (g + ng - 1) % ng, 0)),
            pl.BlockSpec((1, 1, N_BUCKETS), lambda i, g: (i, 0, 0)),
            pl.BlockSpec((1, N_BUCKETS, 1), lambda i, g: (i, 0, 0)),
        ],
        out_specs=[
            pl.BlockSpec((1, GRP * CHUNK, 2 * DH), lambda i, g: (i, g, 0)),
        ],
        out_shape=[
            jax.ShapeDtypeStruct((inst, S, 2 * DH), jnp.float32),
        ],
    )(a_qv, a_qv, off_row, off_col)

    (olr,) = _sc_gather([ol_s.reshape(inst * S, 2 * DH)], u_idx.reshape(-1))
    olr = olr.reshape(N_HASHES, T, H, 2 * DH)
    o_all = olr[..., :DH]
    lse_all = olr[..., DH]
    w = jax.nn.softmax(lse_all, axis=0)[..., None]
    o_comb = jnp.sum(o_all * w, axis=0).reshape(T, D_MODEL)

    y1, y2 = pl.pallas_call(
        _out_ff_body,
        grid=(T // TOK_BLK_C,),
        in_specs=[
            pl.BlockSpec((TOK_BLK_C, D_MODEL), lambda i: (i, 0)),
            pl.BlockSpec((TOK_BLK_C, D_MODEL), lambda i: (i, 0)),
            pl.BlockSpec((TOK_BLK_C, D_MODEL), lambda i: (i, 0)),
            full(D_MODEL, D_MODEL),
            full(1, D_MODEL), full(1, D_MODEL),
            full(D_MODEL, D_FF), full(1, D_FF),
            full(D_FF, D_MODEL), full(1, D_MODEL),
        ],
        out_specs=[
            pl.BlockSpec((TOK_BLK_C, D_MODEL), lambda i: (i, 0)),
            pl.BlockSpec((TOK_BLK_C, D_MODEL), lambda i: (i, 0)),
        ],
        out_shape=[
            jax.ShapeDtypeStruct((T, D_MODEL), jnp.float32),
            jax.ShapeDtypeStruct((T, D_MODEL), jnp.float32),
        ],
    )(o_comb, x1.reshape(T, D_MODEL), x2f, Wo, row(ln2_g), row(ln2_b),
      W1, row(b1), W2, row(b2))

    return (y1.reshape(B, S, D_MODEL), y2.reshape(B, S, D_MODEL))


# speedup vs baseline: 6.5008x; 1.0104x over previous
import functools

import jax
import jax.numpy as jnp
from jax import lax
from jax.experimental import pallas as pl
from jax.experimental.pallas import tpu as pltpu
from jax.experimental.pallas import tpu_sc as plsc

D_MODEL = 1024
D_FF = 4096
H = 16
DH = 64
N_BUCKETS = 64
N_HASHES = 4
CHUNK = 64
GRP = 8
TOK_BLK = 512
TOK_BLK_C = 256


def _proj_body(x_ref, g_ref, b_ref, wqk_ref, wv_ref, rot_ref,
               qkv_ref, rt_ref):
    x = x_ref[...]
    m = jnp.mean(x, axis=-1, keepdims=True)
    xc = x - m
    var = jnp.mean(xc * xc, axis=-1, keepdims=True)
    xn = xc * lax.rsqrt(var + 1e-5) * g_ref[...] + b_ref[...]
    qk = jnp.dot(xn, wqk_ref[...], preferred_element_type=jnp.float32)
    v = jnp.dot(xn, wv_ref[...], preferred_element_type=jnp.float32)
    rt_ref[...] = jnp.dot(qk, rot_ref[...], preferred_element_type=jnp.float32)
    parts = []
    for h in range(H):
        parts.append(qk[:, h * DH:(h + 1) * DH])
        parts.append(v[:, h * DH:(h + 1) * DH])
    qkv_ref[...] = jnp.concatenate(parts, axis=1)


def _csort_body(b_ref, inv_ref, offr_ref, offc_ref):
    bcol = b_ref[0]
    s_len = bcol.shape[0]
    iota_row = lax.broadcasted_iota(jnp.int32, (1, N_BUCKETS), 1).astype(jnp.float32)
    onehot = (bcol == iota_row).astype(jnp.float32)
    pre_incl = onehot
    k = 1
    while k < s_len:
        shifted = jnp.concatenate(
            [jnp.zeros((k, N_BUCKETS), jnp.float32), pre_incl[:s_len - k]],
            axis=0)
        pre_incl = pre_incl + shifted
        k *= 2
    pre = pre_incl - onehot
    tot_row = pre_incl[s_len - 1:s_len, :]
    bi0 = lax.broadcasted_iota(jnp.int32, (N_BUCKETS, N_BUCKETS), 0)
    bi1 = lax.broadcasted_iota(jnp.int32, (N_BUCKETS, N_BUCKETS), 1)
    upper = (bi0 < bi1).astype(jnp.float32)
    off_row = jnp.dot(tot_row, upper, preferred_element_type=jnp.float32)
    tot_col = lax.dot_general(onehot, jnp.ones((s_len, 1), jnp.float32),
                              (((0,), (0,)), ((), ())),
                              preferred_element_type=jnp.float32)
    off_col = jnp.dot((bi0 > bi1).astype(jnp.float32), tot_col,
                      preferred_element_type=jnp.float32)
    inv_ref[0] = jnp.sum(onehot * (pre + off_row), axis=-1, keepdims=True)
    offr_ref[0] = off_row
    offc_ref[0] = off_col


def _attn_body(qv_ref, qvp_ref, offr_ref, offc_ref, o_ref):
    W = GRP * CHUNK
    KW = W + CHUNK
    qvfull = qv_ref[0].reshape(W, 2 * DH)
    qvprev = qvp_ref[0, 0]
    off_row = offr_ref[0]
    off_col = offc_ref[0]
    g = pl.program_id(1)
    nc_total = pl.num_programs(1) * GRP
    base0 = g * W
    pidx = lax.rem(g * GRP + nc_total - 1, nc_total)
    qv_ext = jnp.concatenate([qvprev, qvfull], axis=0)
    k = qv_ext[:, :DH]
    knorm = k * (1.0 / (jnp.sqrt(
        jnp.sum(k * k, axis=-1, keepdims=True)) + 1e-6))
    q = qvfull[:, :DH]
    dots = lax.dot_general(q, knorm, (((1,), (1,)), ((), ())),
                           preferred_element_type=jnp.float32)
    dots = dots * (1.0 / (float(DH) ** 0.5))
    ri = lax.broadcasted_iota(jnp.int32, (W, KW), 0)
    ci = lax.broadcasted_iota(jnp.int32, (W, KW), 1)
    kc = lax.div(ci, CHUNK)
    qc = lax.div(ri, CHUNK)
    band = (kc >= qc) & (kc <= qc + 1)
    self_m = ci == ri + CHUNK
    qslot = (base0 + lax.broadcasted_iota(jnp.int32, (W, 1), 0)
             ).astype(jnp.float32)
    bq = jnp.sum((qslot >= off_row).astype(jnp.float32),
                 axis=-1, keepdims=True)
    ci_row = lax.broadcasted_iota(jnp.int32, (1, KW), 1)
    kslot = jnp.where(ci_row < CHUNK, pidx * CHUNK + ci_row,
                      base0 + ci_row - CHUNK).astype(jnp.float32)
    b_e = jnp.sum((kslot >= off_col).astype(jnp.float32),
                  axis=0, keepdims=True)
    dots = jnp.where(band & (bq == b_e), dots, -1e9)
    dots = jnp.where(self_m, -1e5, dots)
    m = jnp.max(dots, axis=-1, keepdims=True)
    e = jnp.exp(dots - m)
    s = jnp.sum(e, axis=-1, keepdims=True)
    o = jnp.dot(e, qv_ext[:, DH:],
                preferred_element_type=jnp.float32) / s
    lse = m + jnp.log(s)
    o_ref[0] = jnp.concatenate(
        [o, lse, jnp.zeros((W, DH - 1), jnp.float32)], axis=1)


def _out_ff_body(o_ref, x1_ref, x2_ref, wo_ref, g_ref, b_ref,
                 w1_ref, b1_ref, w2_ref, b2_ref, y1_ref, y2_ref):
    y1 = x1_ref[...] + jnp.dot(o_ref[...], wo_ref[...],
                               preferred_element_type=jnp.float32)
    y1_ref[...] = y1
    m = jnp.mean(y1, axis=-1, keepdims=True)
    xc = y1 - m
    var = jnp.mean(xc * xc, axis=-1, keepdims=True)
    t = xc * lax.rsqrt(var + 1e-5) * g_ref[...] + b_ref[...]
    h = jax.nn.gelu(jnp.dot(t, w1_ref[...],
                            preferred_element_type=jnp.float32) + b1_ref[...])
    y2_ref[...] = x2_ref[...] + jnp.dot(h, w2_ref[...],
                                        preferred_element_type=jnp.float32) + b2_ref[...]


_SC_NC, _SC_NS = 2, 16
_SC_NW = _SC_NC * _SC_NS
_GCHK = 128


def _sc_gather(tables, idx):
    M = idx.shape[0]
    W = tables[0].shape[1]
    per_w = M // _SC_NW
    n_it = per_w // _GCHK
    nt = len(tables)
    mesh = plsc.VectorSubcoreMesh(core_axis_name="c", subcore_axis_name="s",
                                  num_cores=_SC_NC, num_subcores=_SC_NS)

    @functools.partial(
        pl.kernel, mesh=mesh,
        out_type=[jax.ShapeDtypeStruct((M, W), jnp.float32)
                  for _ in range(nt)],
        scratch_types=(
            [pltpu.VMEM((_GCHK,), jnp.int32)]
            + [pltpu.VMEM((_GCHK, W), jnp.float32) for _ in range(nt)]
            + [pltpu.SemaphoreType.DMA for _ in range(nt)]
        ),
    )
    def gk(*refs):
        t_hbm = refs[:nt]
        idx_hbm = refs[nt]
        o_hbm = refs[nt + 1:nt + 1 + nt]
        idx_v = refs[nt + 1 + nt]
        rows = refs[nt + 2 + nt:nt + 2 + 2 * nt]
        sems = refs[nt + 2 + 2 * nt:]
        wid = lax.axis_index("s") * _SC_NC + lax.axis_index("c")
        base = wid * per_w

        def body(j, carry):
            off = base + j * _GCHK
            pltpu.sync_copy(idx_hbm.at[pl.ds(off, _GCHK)], idx_v)
            cps = [pltpu.async_copy(t_hbm[i].at[idx_v], rows[i], sems[i])
                   for i in range(nt)]
            for c in cps:
                c.wait()
            for i in range(nt):
                pltpu.sync_copy(rows[i], o_hbm[i].at[pl.ds(off, _GCHK)])
            return carry

        lax.fori_loop(0, n_it, body, 0)

    out = gk(*tables, idx)
    return list(out) if isinstance(out, (list, tuple)) else [out]


def _sc_scatter(src, idx3):
    W = src.shape[1]
    n_src = src.shape[0]
    n_it = idx3.shape[1]
    M = _SC_NW * n_it * _GCHK
    per_w = n_it * _GCHK
    mesh = plsc.VectorSubcoreMesh(core_axis_name="c", subcore_axis_name="s",
                                  num_cores=_SC_NC, num_subcores=_SC_NS)

    @functools.partial(
        pl.kernel, mesh=mesh,
        out_type=jax.ShapeDtypeStruct((M, W), jnp.float32),
        scratch_types=[
            pltpu.VMEM((_GCHK,), jnp.int32),
            pltpu.VMEM((_GCHK, W), jnp.float32),
            pltpu.SemaphoreType.DMA,
        ],
    )
    def sk(src_hbm, idx_hbm, out_hbm, idx_v, rows_v, sem):
        wid = lax.axis_index("s") * _SC_NC + lax.axis_index("c")
        base = wid * per_w

        def body(j, carry):
            src_off = lax.rem(base + j * _GCHK, n_src)
            pltpu.sync_copy(src_hbm.at[pl.ds(src_off, _GCHK)], rows_v)
            pltpu.sync_copy(idx_hbm.at[wid, j], idx_v)
            pltpu.async_copy(rows_v, out_hbm.at[idx_v], sem).wait()
            return carry

        lax.fori_loop(0, n_it, body, 0)

    return sk(src, idx3)


def _build_rotmat():
    rot = jax.random.normal(jax.random.key(42),
                            (N_HASHES, DH, N_BUCKETS // 2), dtype=jnp.float32)
    eye = jnp.eye(H, dtype=jnp.float32)
    blk = jnp.einsum('gh,rdn->rgdhn', eye, rot)
    return blk.transpose(1, 2, 0, 3, 4).reshape(D_MODEL, N_HASHES * H * 32)


def kernel(x1, x2, Wqk, Wv, Wo, W1, b1, W2, b2, ln1_g, ln1_b, ln2_g, ln2_b):
    B, S, _ = x1.shape
    nc = S // CHUNK
    ng = nc // GRP
    T = B * S
    nblk = T // TOK_BLK
    inst = N_HASHES * B * H

    rotmat = _build_rotmat()
    x2f = x2.reshape(T, D_MODEL)

    row = lambda a: a.reshape(1, -1)
    full = lambda r, c: pl.BlockSpec((r, c), lambda i: (0, 0))
    qkv, rt = pl.pallas_call(
        _proj_body,
        grid=(nblk,),
        in_specs=[
            pl.BlockSpec((TOK_BLK, D_MODEL), lambda i: (i, 0)),
            full(1, D_MODEL), full(1, D_MODEL),
            full(D_MODEL, D_MODEL), full(D_MODEL, D_MODEL),
            full(D_MODEL, N_HASHES * H * 32),
        ],
        out_specs=[
            pl.BlockSpec((TOK_BLK, 2 * D_MODEL), lambda i: (i, 0)),
            pl.BlockSpec((TOK_BLK, N_HASHES * H * 32), lambda i: (i, 0)),
        ],
        out_shape=[
            jax.ShapeDtypeStruct((T, 2 * D_MODEL), jnp.float32),
            jax.ShapeDtypeStruct((T, N_HASHES * H * 32), jnp.float32),
        ],
    )(x2f, row(ln1_g), row(ln1_b), Wqk, Wv, rotmat)

    rt = rt.reshape(B, S, N_HASHES, H, 32)
    rt = jnp.concatenate([rt, -rt], axis=-1)
    buckets = jnp.argmax(rt, axis=-1).astype(jnp.int32)
    bcol = (buckets.transpose(2, 0, 3, 1)
            .reshape(inst, S, 1).astype(jnp.float32))

    inv_col, off_row, off_col = pl.pallas_call(
        _csort_body,
        grid=(inst,),
        in_specs=[pl.BlockSpec((1, S, 1), lambda i: (i, 0, 0))],
        out_specs=[
            pl.BlockSpec((1, S, 1), lambda i: (i, 0, 0)),
            pl.BlockSpec((1, 1, N_BUCKETS), lambda i: (i, 0, 0)),
            pl.BlockSpec((1, N_BUCKETS, 1), lambda i: (i, 0, 0)),
        ],
        out_shape=[
            jax.ShapeDtypeStruct((inst, S, 1), jnp.float32),
            jax.ShapeDtypeStruct((inst, 1, N_BUCKETS), jnp.float32),
            jax.ShapeDtypeStruct((inst, N_BUCKETS, 1), jnp.float32),
        ],
    )(bcol)
    inv = inv_col.reshape(N_HASHES, B, H, S).astype(jnp.int32)

    rbh_base = (jnp.arange(N_HASHES * B * H, dtype=jnp.int32)
                .reshape(N_HASHES, B, H) * S)
    u_idx = (rbh_base[..., None] + inv).transpose(0, 1, 3, 2)
    a_qv = _sc_scatter(qkv.reshape(T * H, 2 * DH),
                       u_idx.reshape(_SC_NW, -1, _GCHK))
    a_qv = a_qv.reshape(inst, nc, CHUNK, 2 * DH)

    (ol_s,) = pl.pallas_call(
        _attn_body,
        grid=(inst, ng),
        in_specs=[
            pl.BlockSpec((1, GRP, CHUNK, 2 * DH), lambda i, g: (i, g, 0, 0)),
            pl.BlockSpec((1, 1, CHUNK, 2 * DH),
                         lambda i, g: (i, (g * GRP + nc - 1) % nc, 0, 0)),
            pl.BlockSpec((1, 1, N_BUCKETS), lambda i, g: (i, 0, 0)),
            pl.BlockSpec((1, N_BUCKETS, 1), lambda i, g: (i, 0, 0)),
        ],
        out_specs=[
            pl.BlockSpec((1, GRP * CHUNK, 2 * DH), lambda i, g: (i, g, 0)),
        ],
        out_shape=[
            jax.ShapeDtypeStruct((inst, S, 2 * DH), jnp.float32),
        ],
    )(a_qv, a_qv, off_row, off_col)

    (olr,) = _sc_gather([ol_s.reshape(inst * S, 2 * DH)], u_idx.reshape(-1))
    olr = olr.reshape(N_HASHES, T, H, 2 * DH)
    o_all = olr[..., :DH]
    lse_all = olr[..., DH]
    w = jax.nn.softmax(lse_all, axis=0)[..., None]
    o_comb = jnp.sum(o_all * w, axis=0).reshape(T, D_MODEL)

    y1, y2 = pl.pallas_call(
        _out_ff_body,
        grid=(T // TOK_BLK_C,),
        in_specs=[
            pl.BlockSpec((TOK_BLK_C, D_MODEL), lambda i: (i, 0)),
            pl.BlockSpec((TOK_BLK_C, D_MODEL), lambda i: (i, 0)),
            pl.BlockSpec((TOK_BLK_C, D_MODEL), lambda i: (i, 0)),
            full(D_MODEL, D_MODEL),
            full(1, D_MODEL), full(1, D_MODEL),
            full(D_MODEL, D_FF), full(1, D_FF),
            full(D_FF, D_MODEL), full(1, D_MODEL),
        ],
        out_specs=[
            pl.BlockSpec((TOK_BLK_C, D_MODEL), lambda i: (i, 0)),
            pl.BlockSpec((TOK_BLK_C, D_MODEL), lambda i: (i, 0)),
        ],
        out_shape=[
            jax.ShapeDtypeStruct((T, D_MODEL), jnp.float32),
            jax.ShapeDtypeStruct((T, D_MODEL), jnp.float32),
        ],
    )(o_comb, x1.reshape(T, D_MODEL), x2f, Wo, row(ln2_g), row(ln2_b),
      W1, row(b1), W2, row(b2))

    return (y1.reshape(B, S, D_MODEL), y2.reshape(B, S, D_MODEL))


# speedup vs baseline: 7.5559x; 1.1623x over previous
import functools

import jax
import jax.numpy as jnp
from jax import lax
from jax.experimental import pallas as pl
from jax.experimental.pallas import tpu as pltpu
from jax.experimental.pallas import tpu_sc as plsc

D_MODEL = 1024
D_FF = 4096
H = 16
DH = 64
N_BUCKETS = 64
N_HASHES = 4
CHUNK = 64
GRP = 8
TOK_BLK = 512
TOK_BLK_C = 256


def _proj_body(x_ref, g_ref, b_ref, wqk_ref, wv_ref, rot_ref,
               qkv_ref, rt_ref):
    x = x_ref[...]
    m = jnp.mean(x, axis=-1, keepdims=True)
    xc = x - m
    var = jnp.mean(xc * xc, axis=-1, keepdims=True)
    xn = xc * lax.rsqrt(var + 1e-5) * g_ref[...] + b_ref[...]
    qk = jnp.dot(xn, wqk_ref[...], preferred_element_type=jnp.float32)
    v = jnp.dot(xn, wv_ref[...], preferred_element_type=jnp.float32)
    rt_ref[...] = jnp.dot(qk, rot_ref[...], preferred_element_type=jnp.float32)
    parts = []
    for h in range(H):
        parts.append(qk[:, h * DH:(h + 1) * DH])
        parts.append(v[:, h * DH:(h + 1) * DH])
    qkv_ref[...] = jnp.concatenate(parts, axis=1)


def _csort_body(b_ref, inv_ref, offr_ref, offc_ref):
    bcol = b_ref[0]
    s_len = bcol.shape[0]
    iota_row = lax.broadcasted_iota(jnp.int32, (1, N_BUCKETS), 1).astype(jnp.float32)
    onehot = (bcol == iota_row).astype(jnp.float32)
    pre_incl = onehot
    k = 1
    while k < s_len:
        shifted = jnp.concatenate(
            [jnp.zeros((k, N_BUCKETS), jnp.float32), pre_incl[:s_len - k]],
            axis=0)
        pre_incl = pre_incl + shifted
        k *= 2
    pre = pre_incl - onehot
    tot_row = pre_incl[s_len - 1:s_len, :]
    bi0 = lax.broadcasted_iota(jnp.int32, (N_BUCKETS, N_BUCKETS), 0)
    bi1 = lax.broadcasted_iota(jnp.int32, (N_BUCKETS, N_BUCKETS), 1)
    upper = (bi0 < bi1).astype(jnp.float32)
    off_row = jnp.dot(tot_row, upper, preferred_element_type=jnp.float32)
    tot_col = lax.dot_general(onehot, jnp.ones((s_len, 1), jnp.float32),
                              (((0,), (0,)), ((), ())),
                              preferred_element_type=jnp.float32)
    off_col = jnp.dot((bi0 > bi1).astype(jnp.float32), tot_col,
                      preferred_element_type=jnp.float32)
    inv_ref[0] = jnp.sum(onehot * (pre + off_row), axis=-1, keepdims=True)
    offr_ref[0] = off_row
    offc_ref[0] = off_col


def _attn_body(qv_ref, qvp_ref, offr_ref, offc_ref, o_ref):
    W = GRP * CHUNK
    KW = W + CHUNK
    qvfull = qv_ref[0].reshape(W, 2 * DH)
    qvprev = qvp_ref[0, 0]
    off_row = offr_ref[0]
    off_col = offc_ref[0]
    g = pl.program_id(1)
    nc_total = pl.num_programs(1) * GRP
    base0 = g * W
    pidx = lax.rem(g * GRP + nc_total - 1, nc_total)
    qv_ext = jnp.concatenate([qvprev, qvfull], axis=0)
    k = qv_ext[:, :DH]
    knorm = k * (1.0 / (jnp.sqrt(
        jnp.sum(k * k, axis=-1, keepdims=True)) + 1e-6))
    q = qvfull[:, :DH]
    dots = lax.dot_general(q, knorm, (((1,), (1,)), ((), ())),
                           preferred_element_type=jnp.float32)
    dots = dots * (1.0 / (float(DH) ** 0.5))
    ri = lax.broadcasted_iota(jnp.int32, (W, KW), 0)
    ci = lax.broadcasted_iota(jnp.int32, (W, KW), 1)
    kc = lax.div(ci, CHUNK)
    qc = lax.div(ri, CHUNK)
    band = (kc >= qc) & (kc <= qc + 1)
    self_m = ci == ri + CHUNK
    qslot = (base0 + lax.broadcasted_iota(jnp.int32, (W, 1), 0)
             ).astype(jnp.float32)
    bq = jnp.sum((qslot >= off_row).astype(jnp.float32),
                 axis=-1, keepdims=True)
    ci_row = lax.broadcasted_iota(jnp.int32, (1, KW), 1)
    kslot = jnp.where(ci_row < CHUNK, pidx * CHUNK + ci_row,
                      base0 + ci_row - CHUNK).astype(jnp.float32)
    b_e = jnp.sum((kslot >= off_col).astype(jnp.float32),
                  axis=0, keepdims=True)
    dots = jnp.where(band & (bq == b_e), dots, -1e9)
    dots = jnp.where(self_m, -1e5, dots)
    m = jnp.max(dots, axis=-1, keepdims=True)
    e = jnp.exp(dots - m)
    s = jnp.sum(e, axis=-1, keepdims=True)
    o = jnp.dot(e, qv_ext[:, DH:],
                preferred_element_type=jnp.float32) / s
    lse = m + jnp.log(s)
    o_ref[0] = jnp.concatenate(
        [o, lse, jnp.zeros((W, DH - 1), jnp.float32)], axis=1)


def _out_ff_body(o_ref, x1_ref, x2_ref, wo_ref, g_ref, b_ref,
                 w1_ref, b1_ref, w2_ref, b2_ref, y1_ref, y2_ref):
    y1 = x1_ref[...] + jnp.dot(o_ref[...], wo_ref[...],
                               preferred_element_type=jnp.float32)
    y1_ref[...] = y1
    m = jnp.mean(y1, axis=-1, keepdims=True)
    xc = y1 - m
    var = jnp.mean(xc * xc, axis=-1, keepdims=True)
    t = xc * lax.rsqrt(var + 1e-5) * g_ref[...] + b_ref[...]
    h = jax.nn.gelu(jnp.dot(t, w1_ref[...],
                            preferred_element_type=jnp.float32) + b1_ref[...])
    y2_ref[...] = x2_ref[...] + jnp.dot(h, w2_ref[...],
                                        preferred_element_type=jnp.float32) + b2_ref[...]


_SC_NC, _SC_NS = 2, 16
_SC_NW = _SC_NC * _SC_NS
_GCHK = 128


def _sc_gather(tables, idx):
    M = idx.shape[0]
    W = tables[0].shape[1]
    per_w = M // _SC_NW
    n_it = per_w // _GCHK
    nt = len(tables)
    mesh = plsc.VectorSubcoreMesh(core_axis_name="c", subcore_axis_name="s",
                                  num_cores=_SC_NC, num_subcores=_SC_NS)

    @functools.partial(
        pl.kernel, mesh=mesh,
        out_type=[jax.ShapeDtypeStruct((M, W), jnp.float32)
                  for _ in range(nt)],
        scratch_types=(
            [pltpu.VMEM((_GCHK,), jnp.int32)]
            + [pltpu.VMEM((_GCHK, W), jnp.float32) for _ in range(nt)]
            + [pltpu.SemaphoreType.DMA for _ in range(nt)]
        ),
    )
    def gk(*refs):
        t_hbm = refs[:nt]
        idx_hbm = refs[nt]
        o_hbm = refs[nt + 1:nt + 1 + nt]
        idx_v = refs[nt + 1 + nt]
        rows = refs[nt + 2 + nt:nt + 2 + 2 * nt]
        sems = refs[nt + 2 + 2 * nt:]
        wid = lax.axis_index("s") * _SC_NC + lax.axis_index("c")
        base = wid * per_w

        def body(j, carry):
            off = base + j * _GCHK
            pltpu.sync_copy(idx_hbm.at[pl.ds(off, _GCHK)], idx_v)
            cps = [pltpu.async_copy(t_hbm[i].at[idx_v], rows[i], sems[i])
                   for i in range(nt)]
            for c in cps:
                c.wait()
            for i in range(nt):
                pltpu.sync_copy(rows[i], o_hbm[i].at[pl.ds(off, _GCHK)])
            return carry

        lax.fori_loop(0, n_it, body, 0)

    out = gk(*tables, idx)
    return list(out) if isinstance(out, (list, tuple)) else [out]


def _sc_scatter(src, idx3):
    W = src.shape[1]
    n_src = src.shape[0]
    n_it = idx3.shape[1]
    M = _SC_NW * n_it * _GCHK
    per_w = n_it * _GCHK
    mesh = plsc.VectorSubcoreMesh(core_axis_name="c", subcore_axis_name="s",
                                  num_cores=_SC_NC, num_subcores=_SC_NS)

    @functools.partial(
        pl.kernel, mesh=mesh,
        out_type=jax.ShapeDtypeStruct((M, W), jnp.float32),
        scratch_types=[
            pltpu.VMEM((_GCHK,), jnp.int32),
            pltpu.VMEM((_GCHK, W), jnp.float32),
            pltpu.SemaphoreType.DMA,
        ],
    )
    def sk(src_hbm, idx_hbm, out_hbm, idx_v, rows_v, sem):
        wid = lax.axis_index("s") * _SC_NC + lax.axis_index("c")
        base = wid * per_w

        def body(j, carry):
            src_off = lax.rem(base + j * _GCHK, n_src)
            pltpu.sync_copy(src_hbm.at[pl.ds(src_off, _GCHK)], rows_v)
            pltpu.sync_copy(idx_hbm.at[wid, j], idx_v)
            pltpu.async_copy(rows_v, out_hbm.at[idx_v], sem).wait()
            return carry

        lax.fori_loop(0, n_it, body, 0)

    return sk(src, idx3)


def _build_rotmat():
    rot = jax.random.normal(jax.random.key(42),
                            (N_HASHES, DH, N_BUCKETS // 2), dtype=jnp.float32)
    eye = jnp.eye(H, dtype=jnp.float32)
    blk = jnp.einsum('gh,rdn->rgdhn', eye, rot)
    return blk.transpose(1, 2, 0, 3, 4).reshape(D_MODEL, N_HASHES * H * 32)


def kernel(x1, x2, Wqk, Wv, Wo, W1, b1, W2, b2, ln1_g, ln1_b, ln2_g, ln2_b):
    B, S, _ = x1.shape
    nc = S // CHUNK
    ng = nc // GRP
    T = B * S
    nblk = T // TOK_BLK
    inst = N_HASHES * B * H

    rotmat = _build_rotmat()
    x2f = x2.reshape(T, D_MODEL)

    row = lambda a: a.reshape(1, -1)
    full = lambda r, c: pl.BlockSpec((r, c), lambda i: (0, 0))
    qkv, rt = pl.pallas_call(
        _proj_body,
        grid=(nblk,),
        in_specs=[
            pl.BlockSpec((TOK_BLK, D_MODEL), lambda i: (i, 0)),
            full(1, D_MODEL), full(1, D_MODEL),
            full(D_MODEL, D_MODEL), full(D_MODEL, D_MODEL),
            full(D_MODEL, N_HASHES * H * 32),
        ],
        out_specs=[
            pl.BlockSpec((TOK_BLK, 2 * D_MODEL), lambda i: (i, 0)),
            pl.BlockSpec((TOK_BLK, N_HASHES * H * 32), lambda i: (i, 0)),
        ],
        out_shape=[
            jax.ShapeDtypeStruct((T, 2 * D_MODEL), jnp.float32),
            jax.ShapeDtypeStruct((T, N_HASHES * H * 32), jnp.float32),
        ],
    )(x2f, row(ln1_g), row(ln1_b), Wqk, Wv, rotmat)

    rt = rt.reshape(B, S, N_HASHES, H, 32)
    rt = jnp.concatenate([rt, -rt], axis=-1)
    buckets = jnp.argmax(rt, axis=-1).astype(jnp.int32)
    bcol = (buckets.transpose(2, 0, 3, 1)
            .reshape(inst, S, 1).astype(jnp.float32))

    inv_col, off_row, off_col = pl.pallas_call(
        _csort_body,
        grid=(inst,),
        in_specs=[pl.BlockSpec((1, S, 1), lambda i: (i, 0, 0))],
        out_specs=[
            pl.BlockSpec((1, S, 1), lambda i: (i, 0, 0)),
            pl.BlockSpec((1, 1, N_BUCKETS), lambda i: (i, 0, 0)),
            pl.BlockSpec((1, N_BUCKETS, 1), lambda i: (i, 0, 0)),
        ],
        out_shape=[
            jax.ShapeDtypeStruct((inst, S, 1), jnp.float32),
            jax.ShapeDtypeStruct((inst, 1, N_BUCKETS), jnp.float32),
            jax.ShapeDtypeStruct((inst, N_BUCKETS, 1), jnp.float32),
        ],
    )(bcol)
    inv = inv_col.reshape(N_HASHES, B, H, S).astype(jnp.int32)

    rbh_base = (jnp.arange(N_HASHES * B * H, dtype=jnp.int32)
                .reshape(N_HASHES, B, H) * S)
    u_idx = (rbh_base[..., None] + inv).transpose(0, 1, 3, 2)

    inst_r = B * H
    qkv_flat = qkv.reshape(T * H, 2 * DH)
    olr_rounds = []
    for r in range(N_HASHES):
        u_r = u_idx[r] - (r * inst_r * S)
        a_qv = _sc_scatter(qkv_flat, u_r.reshape(_SC_NW, -1, _GCHK))
        a_qv = a_qv.reshape(inst_r, nc, CHUNK, 2 * DH)
        (ol_s,) = pl.pallas_call(
            _attn_body,
            grid=(inst_r, ng),
            in_specs=[
                pl.BlockSpec((1, GRP, CHUNK, 2 * DH),
                             lambda i, g: (i, g, 0, 0)),
                pl.BlockSpec((1, 1, CHUNK, 2 * DH),
                             lambda i, g: (i, (g * GRP + nc - 1) % nc, 0, 0)),
                pl.BlockSpec((1, 1, N_BUCKETS), lambda i, g: (i, 0, 0)),
                pl.BlockSpec((1, N_BUCKETS, 1), lambda i, g: (i, 0, 0)),
            ],
            out_specs=[
                pl.BlockSpec((1, GRP * CHUNK, 2 * DH), lambda i, g: (i, g, 0)),
            ],
            out_shape=[
                jax.ShapeDtypeStruct((inst_r, S, 2 * DH), jnp.float32),
            ],
        )(a_qv, a_qv, off_row[r * inst_r:(r + 1) * inst_r],
          off_col[r * inst_r:(r + 1) * inst_r])
        (ol_u,) = _sc_gather([ol_s.reshape(inst_r * S, 2 * DH)],
                             u_r.reshape(-1))
        olr_rounds.append(ol_u)
    olr = jnp.stack(olr_rounds).reshape(N_HASHES, T, H, 2 * DH)
    o_all = olr[..., :DH]
    lse_all = olr[..., DH]
    w = jax.nn.softmax(lse_all, axis=0)[..., None]
    o_comb = jnp.sum(o_all * w, axis=0).reshape(T, D_MODEL)

    y1, y2 = pl.pallas_call(
        _out_ff_body,
        grid=(T // TOK_BLK_C,),
        in_specs=[
            pl.BlockSpec((TOK_BLK_C, D_MODEL), lambda i: (i, 0)),
            pl.BlockSpec((TOK_BLK_C, D_MODEL), lambda i: (i, 0)),
            pl.BlockSpec((TOK_BLK_C, D_MODEL), lambda i: (i, 0)),
            full(D_MODEL, D_MODEL),
            full(1, D_MODEL), full(1, D_MODEL),
            full(D_MODEL, D_FF), full(1, D_FF),
            full(D_FF, D_MODEL), full(1, D_MODEL),
        ],
        out_specs=[
            pl.BlockSpec((TOK_BLK_C, D_MODEL), lambda i: (i, 0)),
            pl.BlockSpec((TOK_BLK_C, D_MODEL), lambda i: (i, 0)),
        ],
        out_shape=[
            jax.ShapeDtypeStruct((T, D_MODEL), jnp.float32),
            jax.ShapeDtypeStruct((T, D_MODEL), jnp.float32),
        ],
    )(o_comb, x1.reshape(T, D_MODEL), x2f, Wo, row(ln2_g), row(ln2_b),
      W1, row(b1), W2, row(b2))

    return (y1.reshape(B, S, D_MODEL), y2.reshape(B, S, D_MODEL))
